# baseline (speedup 1.0000x reference)
import functools

import jax
import jax.numpy as jnp
from jax import lax
from jax.experimental import pallas as pl
from jax.experimental.pallas import tpu as pltpu
from jax.experimental.pallas import tpu_sc as plsc

NC = 2
NS = 16
NW = NC * NS
LANES = 16

BATCH = 16384
FEAT = 64
B_PER_W = BATCH // NW
CHUNK = 128
NCHUNK = B_PER_W // CHUNK
PAIR_FEAT = 2 * FEAT

RELAYOUT_ROWS = 8000


def _relayout_body(in_ref, out_ref):
    a = in_ref[...].reshape(RELAYOUT_ROWS // 2, 2, FEAT)
    out_ref[...] = jnp.concatenate([a[:, 0, :], a[:, 1, :]], axis=-1)


def _linearize(centers):
    n_rows = centers.shape[0]
    grid = n_rows // RELAYOUT_ROWS
    return pl.pallas_call(
        _relayout_body,
        grid=(grid,),
        in_specs=[pl.BlockSpec((RELAYOUT_ROWS, FEAT), lambda i: (i, 0))],
        out_specs=pl.BlockSpec((RELAYOUT_ROWS // 2, PAIR_FEAT), lambda i: (i, 0)),
        out_shape=jax.ShapeDtypeStruct((n_rows // 2, PAIR_FEAT), jnp.float32),
    )(centers)


def _make_sc_kernel():
    mesh = plsc.VectorSubcoreMesh(core_axis_name="c", subcore_axis_name="s")

    @functools.partial(
        pl.kernel,
        mesh=mesh,
        out_type=jax.ShapeDtypeStruct((NW, LANES), jnp.float32),
        scratch_types=[
            pltpu.VMEM((NCHUNK, CHUNK), jnp.int32),
            pltpu.VMEM((NCHUNK, CHUNK), jnp.int32),
            pltpu.VMEM((2, CHUNK, PAIR_FEAT), jnp.float32),
            pltpu.VMEM((B_PER_W, FEAT), jnp.float32),
            pltpu.VMEM((LANES,), jnp.float32),
            pltpu.SemaphoreType.DMA,
            pltpu.SemaphoreType.DMA,
        ],
    )
    def body(x_hbm, idx_hbm, off_hbm, table_hbm, out_hbm,
             idx_v, off_v, rows_v, x_v, acc_v, sem0, sem1):
        wid = lax.axis_index("s") * NC + lax.axis_index("c")
        base = wid * B_PER_W
        sems = [sem0, sem1]

        pltpu.sync_copy(idx_hbm.at[wid], idx_v)
        copies = [None] * NCHUNK
        for k in range(2):
            copies[k] = pltpu.async_copy(
                table_hbm.at[idx_v.at[k]], rows_v.at[k % 2], sems[k % 2])
        pltpu.sync_copy(off_hbm.at[wid], off_v)
        pltpu.sync_copy(x_hbm.at[pl.ds(base, B_PER_W)], x_v)

        def chunk_sum(k, acc):
            buf = k % 2

            def group(g, acc):
                hvec = off_v[k, pl.ds(g * LANES, LANES)]
                for j in range(LANES):
                    r = g * LANES + j
                    h = hvec[j]
                    for c in range(FEAT // LANES):
                        xa = x_v[k * CHUNK + r, pl.ds(c * LANES, LANES)]
                        ga = rows_v[buf, r, pl.ds(h + c * LANES, LANES)]
                        d = xa - ga
                        acc = acc + d * d
                return acc

            return lax.fori_loop(0, CHUNK // LANES, group, acc)

        acc = jnp.zeros((LANES,), jnp.float32)
        for k in range(NCHUNK):
            copies[k].wait()
            acc = chunk_sum(k, acc)
            if k + 2 < NCHUNK:
                copies[k + 2] = pltpu.async_copy(
                    table_hbm.at[idx_v.at[k + 2]], rows_v.at[k % 2], sems[k % 2])
        acc_v[...] = acc
        pltpu.sync_copy(acc_v, out_hbm.at[wid])

    return body


_sc_loss_partials = _make_sc_kernel()


@jax.jit
def kernel(x, labels, centers):
    batch, feat = x.shape
    labels32 = labels.astype(jnp.int32)
    pair_idx = (labels32 >> 1).reshape(NW, NCHUNK, CHUNK)
    half_off = ((labels32 & 1) * FEAT).reshape(NW, NCHUNK, CHUNK)
    table = _linearize(centers)
    partials = _sc_loss_partials(x, pair_idx, half_off, table)
    return jnp.sum(partials) / 2.0 / batch


# speedup vs baseline: 2.0073x; 2.0073x over previous
import functools

import jax
import jax.numpy as jnp
from jax import lax
from jax.experimental import pallas as pl
from jax.experimental.pallas import tpu as pltpu
from jax.experimental.pallas import tpu_sc as plsc

NC = 2
NS = 16
NW = NC * NS
LANES = 16

BATCH = 16384
FEAT = 64
B_PER_W = BATCH // NW
CHUNK = 128
NCHUNK = B_PER_W // CHUNK
QUEUES = 4


def _make_sc_kernel():
    mesh = plsc.VectorSubcoreMesh(core_axis_name="c", subcore_axis_name="s")

    @functools.partial(
        pl.kernel,
        mesh=mesh,
        out_type=jax.ShapeDtypeStruct((NW, LANES), jnp.float32),
        scratch_types=[
            pltpu.VMEM((NCHUNK, CHUNK), jnp.int32),
            pltpu.VMEM((2, CHUNK, FEAT), jnp.float32),
            pltpu.VMEM((B_PER_W, FEAT), jnp.float32),
            pltpu.VMEM((LANES,), jnp.float32),
        ] + [pltpu.SemaphoreType.DMA] * (2 * QUEUES),
    )
    def body(x_hbm, lab_hbm, table_hbm, out_hbm,
             lab_v, rows_v, x_v, acc_v, *sems):
        wid = lax.axis_index("s") * NC + lax.axis_index("c")
        base = wid * B_PER_W

        pltpu.sync_copy(lab_hbm.at[wid], lab_v)

        def issue_chunk(k):
            buf = k % 2

            def g_body(g, carry):
                lvec = lab_v[k, pl.ds(g * LANES, LANES)]
                for j in range(LANES):
                    s = lvec[j]
                    pltpu.async_copy(
                        table_hbm.at[s],
                        rows_v.at[buf, g * LANES + j],
                        sems[buf * QUEUES + (j % QUEUES)],
                    )
                return carry

            lax.fori_loop(0, CHUNK // LANES, g_body, 0)

        def drain_chunk(k):
            buf = k % 2
            per_q = CHUNK // QUEUES
            for q in range(QUEUES):
                pltpu.make_async_copy(
                    table_hbm.at[pl.ds(0, per_q)],
                    rows_v.at[buf, pl.ds(0, per_q)],
                    sems[buf * QUEUES + q],
                ).wait()

        issue_chunk(0)
        issue_chunk(1)
        pltpu.sync_copy(x_hbm.at[pl.ds(base, B_PER_W)], x_v)

        def chunk_sum(k, acc):
            buf = k % 2

            def row(r, acc):
                for c in range(FEAT // LANES):
                    xa = x_v[k * CHUNK + r, pl.ds(c * LANES, LANES)]
                    ga = rows_v[buf, r, pl.ds(c * LANES, LANES)]
                    d = xa - ga
                    acc = acc + d * d
                return acc

            return lax.fori_loop(0, CHUNK, row, acc)

        acc = jnp.zeros((LANES,), jnp.float32)
        for k in range(NCHUNK):
            drain_chunk(k)
            acc = chunk_sum(k, acc)
            if k + 2 < NCHUNK:
                issue_chunk(k + 2)
        acc_v[...] = acc
        pltpu.sync_copy(acc_v, out_hbm.at[wid])

    return body


_sc_loss_partials = _make_sc_kernel()


@jax.jit
def kernel(x, labels, centers):
    batch, feat = x.shape
    lab = labels.astype(jnp.int32).reshape(NW, NCHUNK, CHUNK)
    partials = _sc_loss_partials(x, lab, centers)
    return jnp.sum(partials) / 2.0 / batch
